# baseline (speedup 1.0000x reference)
import jax
import jax.numpy as jnp
from jax.experimental import pallas as pl


def _transpose_body(x_ref, o_ref):
    o_ref[0] = x_ref[0].T


def kernel(input):
    b, e, h, w = input.shape
    hw = h * w
    x = input.reshape(b, e, hw)
    out = pl.pallas_call(
        _transpose_body,
        grid=(b,),
        in_specs=[pl.BlockSpec((1, e, hw), lambda i: (i, 0, 0))],
        out_specs=pl.BlockSpec((1, hw, e), lambda i: (i, 0, 0)),
        out_shape=jax.ShapeDtypeStruct((b, hw, e), x.dtype),
    )(x)
    length = jnp.full((b,), True, dtype=bool)
    return (out, length)
